# baseline (speedup 1.0000x reference)
import jax
import jax.numpy as jnp
import numpy as np
from jax.experimental import pallas as pl

_NE = 5
_TK = 3
_NC = 10
_B = 128


def _build_w1p(w1):
    w = w1[:, :, 0, :]
    xx = jnp.arange(32)
    cols = []
    for a in (0, 1):
        x = 2 * jnp.arange(13) + a
        jdx = xx[:, None] - x[None, :]
        valid = (jdx >= 0) & (jdx < 3) & (x[None, :] <= 25)
        g = w[:, jnp.clip(jdx, 0, 2), :] * valid[None, :, :, None]
        g = g.reshape(96, 13, 32).reshape(96, 416)
        cols.append(jnp.pad(g, ((0, 0), (0, 96))))
    return jnp.concatenate(cols, axis=1).astype(jnp.bfloat16)


def _build_w2(w2):
    xx = jnp.arange(13)
    Xx = jnp.arange(10)
    jdx = xx[:, None] - Xx[None, :]
    valid = (jdx >= 0) & (jdx < 3)
    blocks = []
    for i in range(3):
        g = w2[i][jnp.clip(jdx, 0, 2)] * valid[:, :, None, None]
        g = jnp.transpose(g, (0, 2, 1, 3))
        halves = []
        for a in (0, 1):
            h = g[:, :, a::2, :].reshape(416, 320)
            halves.append(jnp.pad(h, ((0, 96), (0, 64))))
        blocks.append(jnp.concatenate(halves, axis=1))
    return jnp.concatenate(blocks, axis=0).astype(jnp.bfloat16)


def _build_wc(gate_w, expert_w):
    wfull = jnp.concatenate(
        [gate_w, jnp.transpose(expert_w, (1, 0, 2)).reshape(1600, 50)], axis=1)
    wfull = jnp.pad(wfull, ((0, 0), (0, 73)))
    w3 = wfull.reshape(5, 320, 128)
    w3 = jnp.pad(w3, ((0, 0), (0, 64), (0, 0)))
    return w3.reshape(1920, 128).astype(jnp.bfloat16)


def _build_xcat(inputs):
    x = jnp.pad(inputs[..., 0], ((0, 0), (0, 8), (0, 4)))
    shifts = [jnp.transpose(x[:, i:i + 32, :], (1, 0, 2)) for i in range(3)]
    return jnp.concatenate(shifts, axis=2).reshape(4096, 96).astype(jnp.bfloat16)


def _body(xcat_ref, w1p_ref, w2_ref, wc_ref, b1_ref, b2_ref, b55_ref, out_ref):
    f32 = jnp.float32

    xc = xcat_ref[...]
    c1 = jnp.dot(xc, w1p_ref[...], preferred_element_type=f32)
    c1 = c1 + b1_ref[...]
    p = jnp.maximum(jnp.maximum(c1[:, :512], c1[:, 512:]), 0.0)

    p3 = p.reshape(16, 2, 128, 512)
    q = jnp.maximum(p3[:, 0], p3[:, 1])
    q2 = q.reshape(2048, 512).astype(jnp.bfloat16)

    w2 = w2_ref[...]
    o2 = jnp.dot(q2[0:1408], w2[0:512], preferred_element_type=f32)
    o2 = o2 + jnp.dot(q2[128:1536], w2[512:1024], preferred_element_type=f32)
    o2 = o2 + jnp.dot(q2[256:1664], w2[1024:1536], preferred_element_type=f32)
    o2 = o2 + b2_ref[...]
    p2 = jnp.maximum(jnp.maximum(o2[:, :384], o2[:, 384:]), 0.0)

    p2r = p2[0:1280].reshape(5, 2, 128, 384)
    h2 = jnp.maximum(p2r[:, 0], p2r[:, 1])
    h2b = h2.astype(jnp.bfloat16)

    wc = wc_ref[...]
    out55 = jnp.dot(h2b[0], wc[0:384], preferred_element_type=f32)
    for yy2 in range(1, 5):
        out55 = out55 + jnp.dot(h2b[yy2], wc[384 * yy2:384 * (yy2 + 1)],
                                preferred_element_type=f32)
    out55 = out55 + b55_ref[...]

    g = out55[:, 0:5]
    g = g - jnp.max(g, axis=1, keepdims=True)
    eg = jnp.exp(g)
    gate = eg / jnp.sum(eg, axis=1, keepdims=True)

    acc = jnp.zeros((128, _NC), dtype=f32)
    for e in range(_NE):
        ge = gate[:, e:e + 1]
        better = (gate > ge).astype(f32)
        if e > 0:
            tie_lt = (jnp.arange(5) < e).astype(f32)
            better = better + (gate == ge).astype(f32) * tie_lt[None, :]
        rank = jnp.sum(better, axis=1, keepdims=True)
        keep = (rank < float(_TK)).astype(f32)
        eo = out55[:, 5 + _NC * e: 5 + _NC * (e + 1)]
        acc = acc + keep * ge * eo

    acc = acc - jnp.max(acc, axis=1, keepdims=True)
    ea = jnp.exp(acc)
    out_ref[...] = ea / jnp.sum(ea, axis=1, keepdims=True)


def kernel(inputs, conv1_w, conv1_b, conv2_w, conv2_b, gate_w, gate_b,
           expert_w, expert_b):
    xcat = _build_xcat(inputs)
    w1p = _build_w1p(conv1_w)
    w2 = _build_w2(conv2_w)
    wc = _build_wc(gate_w, expert_w)

    lane_ok1 = (np.arange(512) < 416)
    b1 = jnp.tile(conv1_b, 16)[None, :] * lane_ok1[None, :]
    b1 = jnp.concatenate([b1, b1], axis=1)
    lane_ok2 = (np.arange(384) < 320)
    b2 = jnp.tile(conv2_b, 6)[None, :] * lane_ok2[None, :]
    b2 = jnp.concatenate([b2, b2], axis=1)
    b55 = jnp.pad(jnp.concatenate([gate_b, expert_b.reshape(50)]),
                  (0, 73))[None, :]

    return pl.pallas_call(
        _body,
        out_shape=jax.ShapeDtypeStruct((_B, _NC), jnp.float32),
    )(xcat, w1p, w2, wc, b1.astype(jnp.float32), b2.astype(jnp.float32),
      b55.astype(jnp.float32))


# speedup vs baseline: 1.5587x; 1.5587x over previous
import jax
import jax.numpy as jnp
import numpy as np
from jax.experimental import pallas as pl

_NE = 5
_TK = 3
_NC = 10
_B = 128


def _c1_const():
    c = np.zeros((3, 32, 32), np.float32)
    for j in range(3):
        for g in range(32):
            a, xx1 = divmod(g, 16)
            x = 2 * xx1 + a
            if xx1 <= 12 and x <= 25 and x + j < 32:
                c[j, x + j, g] = 1.0
    return c


def _c2_const():
    c = np.zeros((3, 16, 12), np.float32)
    for j in range(3):
        for g2 in range(12):
            a, xx2 = divmod(g2, 6)
            if xx2 <= 4:
                xx = 2 * xx2 + a + j
                if xx <= 12:
                    c[j, xx, g2] = 1.0
    return c


_C1 = _c1_const()
_C2 = _c2_const()
_B1MASK = np.zeros((1024,), np.float32)
for _g in range(32):
    if _g % 16 <= 12:
        _B1MASK[_g * 32:(_g + 1) * 32] = 1.0
_ONE1 = np.zeros((1024,), np.float32)
_ONE1[511] = 1.0
_ONE1[1023] = 1.0
_B2MASK = np.zeros((768,), np.float32)
for _g in range(12):
    if _g % 6 <= 4:
        _B2MASK[_g * 64:(_g + 1) * 64] = 1.0
_ONE2 = np.zeros((768,), np.float32)
_ONE2[383] = 1.0
_ONE2[767] = 1.0


def _body(xcat_ref, w1p_ref, w2_ref, wc_ref, out_ref):
    f32 = jnp.float32

    xc = xcat_ref[...]
    c1 = jnp.dot(xc, w1p_ref[...],
                 preferred_element_type=f32)
    p = jnp.maximum(jnp.maximum(c1[:, :512], c1[:, 512:]), 0.0)

    p3 = p.reshape(13, 2, 128, 512)
    q2 = jnp.maximum(p3[:, 0], p3[:, 1]).reshape(1664, 512)
    q2 = q2.astype(jnp.bfloat16)

    w2 = w2_ref[...]
    o2 = jnp.dot(q2[0:1408], w2[0:512], preferred_element_type=f32)
    o2 = o2 + jnp.dot(q2[128:1536], w2[512:1024],
                      preferred_element_type=f32)
    o2 = o2 + jnp.dot(q2[256:1664], w2[1024:1536],
                      preferred_element_type=f32)
    p2 = jnp.maximum(jnp.maximum(o2[:, :384], o2[:, 384:]), 0.0)

    p2r = p2[0:1280].reshape(5, 2, 128, 384)
    h2 = jnp.maximum(p2r[:, 0], p2r[:, 1]).astype(jnp.bfloat16)

    wc = wc_ref[...]
    out55 = jnp.dot(h2[0], wc[0:384], preferred_element_type=f32)
    for yy2 in range(1, 5):
        out55 = out55 + jnp.dot(h2[yy2], wc[384 * yy2:384 * (yy2 + 1)],
                                preferred_element_type=f32)

    g = out55[:, 0:5]
    g = g - jnp.max(g, axis=1, keepdims=True)
    eg = jnp.exp(g)
    gate = eg / jnp.sum(eg, axis=1, keepdims=True)

    acc = jnp.zeros((128, _NC), dtype=f32)
    for e in range(_NE):
        ge = gate[:, e:e + 1]
        better = (gate > ge).astype(f32)
        if e > 0:
            tie_lt = (jnp.arange(5) < e).astype(f32)
            better = better + (gate == ge).astype(f32) * tie_lt[None, :]
        rank = jnp.sum(better, axis=1, keepdims=True)
        keep = (rank < float(_TK)).astype(f32)
        eo = out55[:, 5 + _NC * e: 5 + _NC * (e + 1)]
        acc = acc + keep * ge * eo

    acc = acc - jnp.max(acc, axis=1, keepdims=True)
    ea = jnp.exp(acc)
    out_ref[...] = ea / jnp.sum(ea, axis=1, keepdims=True)


def kernel(inputs, conv1_w, conv1_b, conv2_w, conv2_b, gate_w, gate_b,
           expert_w, expert_b):
    bf16 = jnp.bfloat16

    xp = jnp.pad(inputs[..., 0], ((0, 0), (0, 0), (0, 4)))
    xt = jnp.transpose(xp, (1, 0, 2))
    ones = jnp.ones((26, 128, 32), inputs.dtype)
    xcat = jnp.concatenate([xt[0:26], xt[1:27], xt[2:28], ones],
                           axis=2).reshape(3328, 128).astype(bf16)

    w1e = jnp.einsum('jxg,ijo->ixgo', _C1,
                     conv1_w[:, :, 0, :]).reshape(96, 1024)
    b1row = jnp.tile(conv1_b, 32) * _B1MASK + _ONE1
    w1p = jnp.concatenate(
        [w1e, b1row[None, :], jnp.zeros((31, 1024), w1e.dtype)],
        axis=0).astype(bf16)

    w2e = jnp.einsum('jxg,ijop->ixogp', _C2, conv2_w).reshape(1536, 768)
    b2row = jnp.tile(conv2_b, 12) * _B2MASK + _ONE2
    w2b = jnp.concatenate(
        [w2e[0:511], b2row[None, :], w2e[512:]], axis=0).astype(bf16)

    wfull = jnp.concatenate(
        [gate_w, jnp.transpose(expert_w, (1, 0, 2)).reshape(1600, 50)],
        axis=1)
    wfull = jnp.pad(wfull, ((0, 0), (0, 73))).reshape(5, 320, 128)
    b55row = jnp.pad(jnp.concatenate([gate_b, expert_b.reshape(50)]),
                     (0, 73))
    blk0 = jnp.concatenate(
        [wfull[0], jnp.zeros((63, 128), wfull.dtype), b55row[None, :]],
        axis=0)
    rest = jnp.pad(wfull[1:5], ((0, 0), (0, 64), (0, 0))).reshape(1536, 128)
    wc = jnp.concatenate([blk0, rest], axis=0).astype(bf16)

    return pl.pallas_call(
        _body,
        out_shape=jax.ShapeDtypeStruct((_B, _NC), jnp.float32),
    )(xcat, w1p, w2b, wc)
